# baseline (speedup 1.0000x reference)
import functools

import jax
import jax.numpy as jnp
from jax import lax
from jax.experimental import pallas as pl
from jax.experimental.pallas import tpu as pltpu
from jax.experimental.pallas import tpu_sc as plsc

N = 10000
E = 320000
D = 128
B = 16
OUT = 64

N_PAD = 10240
NW = 32
K = 80
CHUNK = 128
E_PAD = NW * K * CHUNK
W_CH = 8
NWIN = K // W_CH
K_ALL = 2 * K
NWIN0 = K_ALL // W_CH
ROWS_PER_TILE = N_PAD // 16
NEG_INF = float("-inf")


def _sc_mesh():
    return plsc.VectorSubcoreMesh(core_axis_name="c", subcore_axis_name="s")


def _sc_degree(dst2):
    @functools.partial(
        pl.kernel,
        out_type=jax.ShapeDtypeStruct((2 * N_PAD, 16), jnp.float32),
        mesh=_sc_mesh(),
        scratch_types=[
            pltpu.VMEM((K, CHUNK), jnp.int32),
            pltpu.VMEM((CHUNK, 16), jnp.float32),
            pltpu.VMEM((CHUNK, 16), jnp.float32),
            pltpu.VMEM_SHARED((N_PAD, 16), jnp.float32),
        ],
    )
    def k(dst_hbm, out_hbm, dstv, vbuf, zbuf, acc):
        cid = lax.axis_index("c")
        sid = lax.axis_index("s")
        wid = sid * 2 + cid
        lane = lax.iota(jnp.int32, 16)
        e0 = jnp.where(lane == 0, 1.0, 0.0).astype(jnp.float32)
        z16 = jnp.zeros((16,), jnp.float32)

        def fill(j, carry):
            vbuf[j, :] = e0
            zbuf[j, :] = z16
            return carry

        lax.fori_loop(0, CHUNK, fill, 0)
        row0 = sid * ROWS_PER_TILE
        for r in range(ROWS_PER_TILE // CHUNK):
            pltpu.sync_copy(zbuf, acc.at[pl.ds(row0 + r * CHUNK, CHUNK)])
        plsc.subcore_barrier()
        pltpu.sync_copy(dst_hbm.at[wid], dstv)

        def body(kk, carry):
            pltpu.sync_copy(vbuf, acc.at[dstv.at[kk]], add=True)
            return carry

        lax.fori_loop(0, K, body, 0)
        plsc.subcore_barrier()
        out0 = cid * N_PAD + row0
        for r in range(ROWS_PER_TILE // CHUNK):
            pltpu.sync_copy(
                acc.at[pl.ds(row0 + r * CHUNK, CHUNK)],
                out_hbm.at[pl.ds(out0 + r * CHUNK, CHUNK)],
            )

    return k(dst2)


def _sc_edge_pass(y, src4, dst4):
    @functools.partial(
        pl.kernel,
        out_type=jax.ShapeDtypeStruct((N_PAD, D), jnp.float32),
        mesh=_sc_mesh(),
        scratch_types=[
            pltpu.VMEM((W_CH, CHUNK), jnp.int32),
            pltpu.VMEM((W_CH, CHUNK), jnp.int32),
            pltpu.VMEM((CHUNK, D), jnp.float32),
            pltpu.VMEM((CHUNK, D), jnp.float32),
            pltpu.VMEM_SHARED((N_PAD, D), jnp.float32),
            pltpu.SemaphoreType.DMA,
            pltpu.SemaphoreType.DMA,
        ],
    )
    def k(y_hbm, src_hbm, dst_hbm, out_hbm,
          srcv, dstv, rbuf0, rbuf1, acc, sem0, sem1):
        cid = lax.axis_index("c")
        sid = lax.axis_index("s")

        @pl.when(cid == 0)
        def _():
            z16 = jnp.zeros((16,), jnp.float32)

            def fill(j, carry):
                for i in range(D // 16):
                    rbuf0[j, pl.ds(i * 16, 16)] = z16
                return carry

            lax.fori_loop(0, CHUNK, fill, 0)
            row0 = sid * ROWS_PER_TILE
            for r in range(ROWS_PER_TILE // CHUNK):
                pltpu.sync_copy(rbuf0,
                                acc.at[pl.ds(row0 + r * CHUNK, CHUNK)])
            plsc.subcore_barrier()

            def window(win, carry):
                w_lin = sid * NWIN0 + win
                pltpu.sync_copy(src_hbm.at[w_lin], srcv)
                pltpu.sync_copy(dst_hbm.at[w_lin], dstv)
                pltpu.async_copy(y_hbm.at[srcv.at[0]], rbuf0, sem0)
                G = W_CH // 2

                def body(g, c2):
                    k0 = 2 * g
                    k1 = k0 + 1
                    pltpu.async_copy(y_hbm.at[srcv.at[k1]], rbuf1, sem1)
                    pltpu.make_async_copy(
                        y_hbm.at[srcv.at[k0]], rbuf0, sem0).wait()
                    pltpu.sync_copy(rbuf0, acc.at[dstv.at[k0]], add=True)

                    @pl.when(g < G - 1)
                    def _():
                        pltpu.async_copy(
                            y_hbm.at[srcv.at[k0 + 2]], rbuf0, sem0)

                    pltpu.make_async_copy(
                        y_hbm.at[srcv.at[k1]], rbuf1, sem1).wait()
                    pltpu.sync_copy(rbuf1, acc.at[dstv.at[k1]], add=True)
                    return c2

                lax.fori_loop(0, G, body, 0)
                return carry

            lax.fori_loop(0, NWIN0, window, 0)
            plsc.subcore_barrier()
            for r in range(ROWS_PER_TILE // CHUNK):
                pltpu.sync_copy(
                    acc.at[pl.ds(row0 + r * CHUNK, CHUNK)],
                    out_hbm.at[pl.ds(row0 + r * CHUNK, CHUNK)],
                )

    return k(y, src4, dst4)


def _tc_prep(x_pad, W0, degs):
    nb = N_PAD // 128

    def body(x_ref, w_ref, d0_ref, d1_ref, y_ref, dv_ref):
        deg = jnp.sum(d0_ref[...] + d1_ref[...], axis=1, keepdims=True) + 1.0
        dinvb = jnp.broadcast_to(lax.rsqrt(deg), (128, D))
        dv_ref[...] = dinvb
        y_ref[...] = jnp.dot(x_ref[...], w_ref[...],
                             preferred_element_type=jnp.float32) * dinvb

    return pl.pallas_call(
        body,
        grid=(nb,),
        in_specs=[
            pl.BlockSpec((128, D), lambda i: (i, 0)),
            pl.BlockSpec((D, D), lambda i: (0, 0)),
            pl.BlockSpec((128, 16), lambda i: (i, 0)),
            pl.BlockSpec((128, 16), lambda i: (i + nb, 0)),
        ],
        out_specs=[
            pl.BlockSpec((128, D), lambda i: (i, 0)),
            pl.BlockSpec((128, D), lambda i: (i, 0)),
        ],
        out_shape=[
            jax.ShapeDtypeStruct((N_PAD, D), jnp.float32),
            jax.ShapeDtypeStruct((N_PAD, D), jnp.float32),
        ],
    )(x_pad, W0, degs, degs)


def _tc_combine(acc, y_prev, dinvb, b2d, W_next):
    nb = N_PAD // 128

    def body(a_ref, y_ref, dv_ref, b_ref, w_ref, o_ref):
        dv = dv_ref[...]
        h = jnp.maximum(
            dv * (a_ref[...] + y_ref[...]) + b_ref[...], 0.0)
        o_ref[...] = jnp.dot(h, w_ref[...],
                             preferred_element_type=jnp.float32) * dv

    return pl.pallas_call(
        body,
        grid=(nb,),
        in_specs=[
            pl.BlockSpec((128, D), lambda i: (i, 0)),
            pl.BlockSpec((128, D), lambda i: (i, 0)),
            pl.BlockSpec((128, D), lambda i: (i, 0)),
            pl.BlockSpec((1, D), lambda i: (0, 0)),
            pl.BlockSpec((D, D), lambda i: (0, 0)),
        ],
        out_specs=pl.BlockSpec((128, D), lambda i: (i, 0)),
        out_shape=jax.ShapeDtypeStruct((N_PAD, D), jnp.float32),
    )(acc, y_prev, dinvb, b2d, W_next)


def _tc_final(acc, y2, dinvb, b2d, batb, Wm1, bm1_2d, Wm2, bm2_2d):
    nb = N_PAD // 128

    def body(a_ref, y_ref, dv_ref, b_ref, bat_ref,
             wm1_ref, bm1_ref, wm2_ref, bm2_ref,
             out_ref, enc_ref, adds, maxs, cnts):
        i = pl.program_id(0)

        @pl.when(i == 0)
        def _():
            adds[...] = jnp.zeros((B, D), jnp.float32)
            cnts[...] = jnp.zeros((B, D), jnp.float32)
            maxs[...] = jnp.full((B, D), NEG_INF, jnp.float32)

        dv = dv_ref[...]
        h = jnp.maximum(
            dv * (a_ref[...] + y_ref[...]) + b_ref[...], 0.0)
        batm = bat_ref[...]
        brow = batm[0:1, :]
        segs = lax.broadcasted_iota(jnp.int32, (B, 128), 0)
        maskf = (segs == jnp.broadcast_to(brow, (B, 128))).astype(jnp.float32)
        adds[...] += jnp.dot(maskf, h, preferred_element_type=jnp.float32)
        cnts[...] += jnp.broadcast_to(
            jnp.sum(maskf, axis=1, keepdims=True), (B, D))
        for s in range(B):
            hm = jnp.where(batm == s, h, NEG_INF)
            maxs[s:s + 1, :] = jnp.maximum(
                maxs[s:s + 1, :], jnp.max(hm, axis=0, keepdims=True))

        @pl.when(i == nb - 1)
        def _():
            addv = adds[...]
            mean = addv / jnp.maximum(cnts[...], 1.0)
            enc = jnp.concatenate([addv, mean, maxs[...]], axis=1)
            enc_ref[...] = enc
            z = jnp.maximum(
                jnp.dot(enc, wm1_ref[...],
                        preferred_element_type=jnp.float32) + bm1_ref[...],
                0.0)
            out_ref[...] = jnp.dot(
                z, wm2_ref[...],
                preferred_element_type=jnp.float32) + bm2_ref[...]

    return pl.pallas_call(
        body,
        grid=(nb,),
        in_specs=[
            pl.BlockSpec((128, D), lambda i: (i, 0)),
            pl.BlockSpec((128, D), lambda i: (i, 0)),
            pl.BlockSpec((128, D), lambda i: (i, 0)),
            pl.BlockSpec((1, D), lambda i: (0, 0)),
            pl.BlockSpec((128, 128), lambda i: (i, 0)),
            pl.BlockSpec((3 * D, D), lambda i: (0, 0)),
            pl.BlockSpec((1, D), lambda i: (0, 0)),
            pl.BlockSpec((D, OUT), lambda i: (0, 0)),
            pl.BlockSpec((1, OUT), lambda i: (0, 0)),
        ],
        out_specs=[
            pl.BlockSpec((B, OUT), lambda i: (0, 0)),
            pl.BlockSpec((B, 3 * D), lambda i: (0, 0)),
        ],
        out_shape=[
            jax.ShapeDtypeStruct((B, OUT), jnp.float32),
            jax.ShapeDtypeStruct((B, 3 * D), jnp.float32),
        ],
        scratch_shapes=[
            pltpu.VMEM((B, D), jnp.float32),
            pltpu.VMEM((B, D), jnp.float32),
            pltpu.VMEM((B, D), jnp.float32),
        ],
    )(acc, y2, dinvb, b2d, batb, Wm1, bm1_2d, Wm2, bm2_2d)


def kernel(x, edge_index, batch, W0, b0, W1, b1, W2, b2, Wm1, bm1, Wm2, bm2):
    pad_e = jnp.full((E_PAD - E,), N, dtype=jnp.int32)
    src2 = jnp.concatenate([edge_index[0], pad_e]).reshape(NW, K, CHUNK)
    dst2 = jnp.concatenate([edge_index[1], pad_e]).reshape(NW, K, CHUNK)
    src4 = src2.reshape(NW * NWIN, W_CH, CHUNK)
    dst4 = dst2.reshape(NW * NWIN, W_CH, CHUNK)
    x_pad = jnp.pad(x, ((0, N_PAD - N), (0, 0)))
    batch_pad = jnp.concatenate(
        [batch, jnp.full((N_PAD - N,), B, dtype=jnp.int32)])
    batb = jnp.broadcast_to(batch_pad[:, None], (N_PAD, 128))

    degs = _sc_degree(dst2)
    y0, dinvb = _tc_prep(x_pad, W0, degs)

    acc0 = _sc_edge_pass(y0, src4, dst4)
    y1 = _tc_combine(acc0, y0, dinvb, b0.reshape(1, D), W1)
    acc1 = _sc_edge_pass(y1, src4, dst4)
    y2 = _tc_combine(acc1, y1, dinvb, b1.reshape(1, D), W2)
    acc2 = _sc_edge_pass(y2, src4, dst4)

    out, enc = _tc_final(acc2, y2, dinvb, b2.reshape(1, D), batb,
                         Wm1, bm1.reshape(1, D), Wm2, bm2.reshape(1, OUT))
    return (out, enc)


# speedup vs baseline: 3.2339x; 3.2339x over previous
import functools

import jax
import jax.numpy as jnp
from jax import lax
from jax.experimental import pallas as pl
from jax.experimental.pallas import tpu as pltpu
from jax.experimental.pallas import tpu_sc as plsc

N = 10000
E = 320000
D = 128
B = 16
OUT = 64

N_PAD = 10240
NW = 32
K = 80
CHUNK = 128
E_PAD = NW * K * CHUNK
W_CH = 8
NWIN = K // W_CH
ROWS_PER_TILE = N_PAD // 16
NEG_INF = float("-inf")


def _sc_mesh():
    return plsc.VectorSubcoreMesh(core_axis_name="c", subcore_axis_name="s")


def _sc_degree(dst2):
    @functools.partial(
        pl.kernel,
        out_type=jax.ShapeDtypeStruct((2 * N_PAD, 16), jnp.float32),
        mesh=_sc_mesh(),
        scratch_types=[
            pltpu.VMEM((K, CHUNK), jnp.int32),
            pltpu.VMEM((CHUNK, 16), jnp.float32),
            pltpu.VMEM((CHUNK, 16), jnp.float32),
            pltpu.VMEM_SHARED((N_PAD, 16), jnp.float32),
        ],
    )
    def k(dst_hbm, out_hbm, dstv, vbuf, zbuf, acc):
        cid = lax.axis_index("c")
        sid = lax.axis_index("s")
        wid = sid * 2 + cid
        lane = lax.iota(jnp.int32, 16)
        e0 = jnp.where(lane == 0, 1.0, 0.0).astype(jnp.float32)
        z16 = jnp.zeros((16,), jnp.float32)

        def fill(j, carry):
            vbuf[j, :] = e0
            zbuf[j, :] = z16
            return carry

        lax.fori_loop(0, CHUNK, fill, 0)
        row0 = sid * ROWS_PER_TILE
        for r in range(ROWS_PER_TILE // CHUNK):
            pltpu.sync_copy(zbuf, acc.at[pl.ds(row0 + r * CHUNK, CHUNK)])
        plsc.subcore_barrier()
        pltpu.sync_copy(dst_hbm.at[wid], dstv)

        def body(kk, carry):
            pltpu.sync_copy(vbuf, acc.at[dstv.at[kk]], add=True)
            return carry

        lax.fori_loop(0, K, body, 0)
        plsc.subcore_barrier()
        out0 = cid * N_PAD + row0
        for r in range(ROWS_PER_TILE // CHUNK):
            pltpu.sync_copy(
                acc.at[pl.ds(row0 + r * CHUNK, CHUNK)],
                out_hbm.at[pl.ds(out0 + r * CHUNK, CHUNK)],
            )

    return k(dst2)


def _sc_edge_pass(y, src4, dst4):
    @functools.partial(
        pl.kernel,
        out_type=jax.ShapeDtypeStruct((2 * N_PAD, D), jnp.float32),
        mesh=_sc_mesh(),
        scratch_types=[
            pltpu.VMEM((W_CH, CHUNK), jnp.int32),
            pltpu.VMEM((W_CH, CHUNK), jnp.int32),
            pltpu.VMEM((CHUNK, D), jnp.float32),
            pltpu.VMEM((CHUNK, D), jnp.float32),
            pltpu.VMEM_SHARED((N_PAD, D), jnp.float32),
            pltpu.SemaphoreType.DMA,
            pltpu.SemaphoreType.DMA,
        ],
    )
    def k(y_hbm, src_hbm, dst_hbm, out_hbm,
          srcv, dstv, rbuf0, rbuf1, acc, sem0, sem1):
        cid = lax.axis_index("c")
        sid = lax.axis_index("s")
        wid = sid * 2 + cid
        z16 = jnp.zeros((16,), jnp.float32)

        def fill(j, carry):
            for i in range(D // 16):
                rbuf0[j, pl.ds(i * 16, 16)] = z16
            return carry

        lax.fori_loop(0, CHUNK, fill, 0)
        row0 = sid * ROWS_PER_TILE
        for r in range(ROWS_PER_TILE // CHUNK):
            pltpu.sync_copy(rbuf0, acc.at[pl.ds(row0 + r * CHUNK, CHUNK)])
        plsc.subcore_barrier()

        def window(win, carry):
            w_lin = wid * NWIN + win
            pltpu.sync_copy(src_hbm.at[w_lin], srcv)
            pltpu.sync_copy(dst_hbm.at[w_lin], dstv)
            pltpu.async_copy(y_hbm.at[srcv.at[0]], rbuf0, sem0)
            G = W_CH // 2

            def body(g, c2):
                k0 = 2 * g
                k1 = k0 + 1
                pltpu.async_copy(y_hbm.at[srcv.at[k1]], rbuf1, sem1)
                pltpu.make_async_copy(
                    y_hbm.at[srcv.at[k0]], rbuf0, sem0).wait()
                pltpu.sync_copy(rbuf0, acc.at[dstv.at[k0]], add=True)

                @pl.when(g < G - 1)
                def _():
                    pltpu.async_copy(y_hbm.at[srcv.at[k0 + 2]], rbuf0, sem0)

                pltpu.make_async_copy(
                    y_hbm.at[srcv.at[k1]], rbuf1, sem1).wait()
                pltpu.sync_copy(rbuf1, acc.at[dstv.at[k1]], add=True)
                return c2

            lax.fori_loop(0, G, body, 0)
            return carry

        lax.fori_loop(0, NWIN, window, 0)
        plsc.subcore_barrier()
        out0 = cid * N_PAD + row0
        for r in range(ROWS_PER_TILE // CHUNK):
            pltpu.sync_copy(
                acc.at[pl.ds(row0 + r * CHUNK, CHUNK)],
                out_hbm.at[pl.ds(out0 + r * CHUNK, CHUNK)],
            )

    return k(y, src4, dst4)


def _tc_prep(x_pad, W0, degs):
    nb = N_PAD // 128

    def body(x_ref, w_ref, d0_ref, d1_ref, y_ref, dv_ref):
        deg = jnp.sum(d0_ref[...] + d1_ref[...], axis=1, keepdims=True) + 1.0
        dinvb = jnp.broadcast_to(lax.rsqrt(deg), (128, D))
        dv_ref[...] = dinvb
        y_ref[...] = jnp.dot(x_ref[...], w_ref[...],
                             preferred_element_type=jnp.float32) * dinvb

    return pl.pallas_call(
        body,
        grid=(nb,),
        in_specs=[
            pl.BlockSpec((128, D), lambda i: (i, 0)),
            pl.BlockSpec((D, D), lambda i: (0, 0)),
            pl.BlockSpec((128, 16), lambda i: (i, 0)),
            pl.BlockSpec((128, 16), lambda i: (i + nb, 0)),
        ],
        out_specs=[
            pl.BlockSpec((128, D), lambda i: (i, 0)),
            pl.BlockSpec((128, D), lambda i: (i, 0)),
        ],
        out_shape=[
            jax.ShapeDtypeStruct((N_PAD, D), jnp.float32),
            jax.ShapeDtypeStruct((N_PAD, D), jnp.float32),
        ],
    )(x_pad, W0, degs, degs)


def _tc_combine(accs, y_prev, dinvb, b2d, W_next):
    nb = N_PAD // 128

    def body(a0_ref, a1_ref, y_ref, dv_ref, b_ref, w_ref, o_ref):
        dv = dv_ref[...]
        h = jnp.maximum(
            dv * (a0_ref[...] + a1_ref[...] + y_ref[...]) + b_ref[...], 0.0)
        o_ref[...] = jnp.dot(h, w_ref[...],
                             preferred_element_type=jnp.float32) * dv

    return pl.pallas_call(
        body,
        grid=(nb,),
        in_specs=[
            pl.BlockSpec((128, D), lambda i: (i, 0)),
            pl.BlockSpec((128, D), lambda i: (i + nb, 0)),
            pl.BlockSpec((128, D), lambda i: (i, 0)),
            pl.BlockSpec((128, D), lambda i: (i, 0)),
            pl.BlockSpec((1, D), lambda i: (0, 0)),
            pl.BlockSpec((D, D), lambda i: (0, 0)),
        ],
        out_specs=pl.BlockSpec((128, D), lambda i: (i, 0)),
        out_shape=jax.ShapeDtypeStruct((N_PAD, D), jnp.float32),
    )(accs, accs, y_prev, dinvb, b2d, W_next)


def _tc_final(accs, y2, dinvb, b2d, batb, Wm1, bm1_2d, Wm2, bm2_2d):
    nb = N_PAD // 128

    def body(a0_ref, a1_ref, y_ref, dv_ref, b_ref, bat_ref,
             wm1_ref, bm1_ref, wm2_ref, bm2_ref,
             out_ref, enc_ref, adds, maxs, cnts):
        i = pl.program_id(0)

        @pl.when(i == 0)
        def _():
            adds[...] = jnp.zeros((B, D), jnp.float32)
            cnts[...] = jnp.zeros((B, D), jnp.float32)
            maxs[...] = jnp.full((B, D), NEG_INF, jnp.float32)

        dv = dv_ref[...]
        h = jnp.maximum(
            dv * (a0_ref[...] + a1_ref[...] + y_ref[...]) + b_ref[...], 0.0)
        batm = bat_ref[...]
        brow = batm[0:1, :]
        segs = lax.broadcasted_iota(jnp.int32, (B, 128), 0)
        maskf = (segs == jnp.broadcast_to(brow, (B, 128))).astype(jnp.float32)
        adds[...] += jnp.dot(maskf, h, preferred_element_type=jnp.float32)
        cnts[...] += jnp.broadcast_to(
            jnp.sum(maskf, axis=1, keepdims=True), (B, D))
        for s in range(B):
            hm = jnp.where(batm == s, h, NEG_INF)
            maxs[s:s + 1, :] = jnp.maximum(
                maxs[s:s + 1, :], jnp.max(hm, axis=0, keepdims=True))

        @pl.when(i == nb - 1)
        def _():
            addv = adds[...]
            mean = addv / jnp.maximum(cnts[...], 1.0)
            enc = jnp.concatenate([addv, mean, maxs[...]], axis=1)
            enc_ref[...] = enc
            z = jnp.maximum(
                jnp.dot(enc, wm1_ref[...],
                        preferred_element_type=jnp.float32) + bm1_ref[...],
                0.0)
            out_ref[...] = jnp.dot(
                z, wm2_ref[...],
                preferred_element_type=jnp.float32) + bm2_ref[...]

    return pl.pallas_call(
        body,
        grid=(nb,),
        in_specs=[
            pl.BlockSpec((128, D), lambda i: (i, 0)),
            pl.BlockSpec((128, D), lambda i: (i + nb, 0)),
            pl.BlockSpec((128, D), lambda i: (i, 0)),
            pl.BlockSpec((128, D), lambda i: (i, 0)),
            pl.BlockSpec((1, D), lambda i: (0, 0)),
            pl.BlockSpec((128, 128), lambda i: (i, 0)),
            pl.BlockSpec((3 * D, D), lambda i: (0, 0)),
            pl.BlockSpec((1, D), lambda i: (0, 0)),
            pl.BlockSpec((D, OUT), lambda i: (0, 0)),
            pl.BlockSpec((1, OUT), lambda i: (0, 0)),
        ],
        out_specs=[
            pl.BlockSpec((B, OUT), lambda i: (0, 0)),
            pl.BlockSpec((B, 3 * D), lambda i: (0, 0)),
        ],
        out_shape=[
            jax.ShapeDtypeStruct((B, OUT), jnp.float32),
            jax.ShapeDtypeStruct((B, 3 * D), jnp.float32),
        ],
        scratch_shapes=[
            pltpu.VMEM((B, D), jnp.float32),
            pltpu.VMEM((B, D), jnp.float32),
            pltpu.VMEM((B, D), jnp.float32),
        ],
    )(accs, accs, y2, dinvb, b2d, batb, Wm1, bm1_2d, Wm2, bm2_2d)


def kernel(x, edge_index, batch, W0, b0, W1, b1, W2, b2, Wm1, bm1, Wm2, bm2):
    pad_e = N + (jnp.arange(E_PAD - E, dtype=jnp.int32) % (N_PAD - N))
    src2 = jnp.concatenate([edge_index[0], pad_e]).reshape(NW, K, CHUNK)
    dst2 = jnp.concatenate([edge_index[1], pad_e]).reshape(NW, K, CHUNK)
    src4 = src2.reshape(NW * NWIN, W_CH, CHUNK)
    dst4 = dst2.reshape(NW * NWIN, W_CH, CHUNK)
    x_pad = jnp.pad(x, ((0, N_PAD - N), (0, 0)))
    batch_pad = jnp.concatenate(
        [batch, jnp.full((N_PAD - N,), B, dtype=jnp.int32)])
    batb = jnp.broadcast_to(batch_pad[:, None], (N_PAD, 128))

    degs = _sc_degree(dst2)
    y0, dinvb = _tc_prep(x_pad, W0, degs)

    acc0 = _sc_edge_pass(y0, src4, dst4)
    y1 = _tc_combine(acc0, y0, dinvb, b0.reshape(1, D), W1)
    acc1 = _sc_edge_pass(y1, src4, dst4)
    y2 = _tc_combine(acc1, y1, dinvb, b1.reshape(1, D), W2)
    acc2 = _sc_edge_pass(y2, src4, dst4)

    out, enc = _tc_final(acc2, y2, dinvb, b2.reshape(1, D), batb,
                         Wm1, bm1.reshape(1, D), Wm2, bm2.reshape(1, OUT))
    return (out, enc)
